# Initial kernel scaffold; baseline (speedup 1.0000x reference)
import jax
import jax.numpy as jnp
from jax.experimental import pallas as pl


def kernel(x, edge_index, W1, b1, W2, b2, Wfc, bfc):
    raise NotImplementedError("write your pallas kernel here")


# speedup vs baseline: 2.8688x; 2.8688x over previous
import jax
import jax.numpy as jnp
from jax.experimental import pallas as pl

N = 7
NP = 8
E = 32
F0 = 224
H1 = 64
H2 = 256
OUT = 576


def _gnn_kernel(ei_ref, xT_ref, w1_ref, b1_ref, w2_ref, b2_ref,
                wfc_ref, bfc_ref, out_ref):
    f32 = jnp.float32
    row = ei_ref[0:1, :E]
    col = ei_ref[1:2, :E]
    nodes = jax.lax.broadcasted_iota(jnp.int32, (NP, 1), 0)
    ohr = (nodes == row).astype(f32)
    ohc = (nodes == col).astype(f32)
    real = (nodes < N).astype(f32)
    deg = jnp.sum(ohc, axis=1, keepdims=True) + real
    dinv = jnp.where(deg > 0, jax.lax.rsqrt(jnp.maximum(deg, 1e-12)), 0.0)
    dinv_row = jnp.sum(ohr * dinv, axis=0, keepdims=True)
    dinv_col = jnp.sum(ohc * dinv, axis=0, keepdims=True)
    norm = dinv_row * dinv_col
    A = jax.lax.dot_general(ohc * norm, ohr,
                            (((1,), (1,)), ((), ())),
                            preferred_element_type=f32)
    eye = (nodes == jax.lax.broadcasted_iota(jnp.int32, (1, NP), 1)).astype(f32)
    A = A + eye * (dinv * dinv) * real

    xw1 = jnp.dot(xT_ref[...], w1_ref[...], preferred_element_type=f32)
    h1 = jax.nn.relu(jnp.dot(A, xw1, preferred_element_type=f32)
                     + b1_ref[...])
    xw2 = jnp.dot(h1, w2_ref[...], preferred_element_type=f32)
    h2 = jax.nn.relu(jnp.dot(A, xw2, preferred_element_type=f32)
                     + b2_ref[...])

    acc = bfc_ref[...]
    for n in range(N):
        acc = acc + jnp.dot(h2[n:n + 1, :], wfc_ref[n],
                            preferred_element_type=f32)
    out_ref[...] = acc


def kernel(x, edge_index, W1, b1, W2, b2, Wfc, bfc):
    xT = jnp.pad(x.T, ((0, NP - N), (0, 0)))
    ei = jnp.pad(edge_index.astype(jnp.int32), ((0, 6), (0, 96)))
    out = pl.pallas_call(
        _gnn_kernel,
        out_shape=jax.ShapeDtypeStruct((1, OUT), jnp.float32),
    )(ei, xT, W1, b1.reshape(1, H1), W2, b2.reshape(1, H2),
      Wfc.reshape(N, H2, OUT), bfc.reshape(1, OUT))
    return out.reshape(24, 24)
